# baseline (speedup 1.0000x reference)
import functools

import jax
import jax.numpy as jnp
from jax import lax
from jax.experimental import pallas as pl
from jax.experimental.pallas import tpu as pltpu
from jax.experimental.pallas import tpu_sc as plsc

_VOCAB = 32000
_D = 2048
_NTOK = 4 * 2048

_NC, _NS = 2, 16
_NW = _NC * _NS

_CHUNKS = (2048, 3072, 3072)
_CH = 16


def _sc_gather(table, idx, tpc):
    mesh = plsc.VectorSubcoreMesh(core_axis_name="c", subcore_axis_name="s")
    b_per_w = tpc // _NW
    nch = b_per_w // _CH

    @functools.partial(
        pl.kernel,
        mesh=mesh,
        out_type=jax.ShapeDtypeStruct((tpc, _D), jnp.float32),
        scratch_types=[
            pltpu.VMEM((b_per_w,), jnp.int32),
            pltpu.VMEM((_CH, _D), jnp.float32),
            pltpu.VMEM((_CH, _D), jnp.float32),
            pltpu.SemaphoreType.DMA,
            pltpu.SemaphoreType.DMA,
            pltpu.SemaphoreType.DMA,
            pltpu.SemaphoreType.DMA,
        ],
    )
    def gather_kernel(table_hbm, idx_hbm, out_hbm, idx_v, rows0, rows1,
                      gs0, gs1, os0, os1):
        wid = lax.axis_index("s") * _NC + lax.axis_index("c")
        base = wid * b_per_w
        bufs, gs, os = (rows0, rows1), (gs0, gs1), (os0, os1)
        pltpu.sync_copy(idx_hbm.at[pl.ds(base, b_per_w)], idx_v)

        def gather_in(j, b):
            return pltpu.make_async_copy(
                table_hbm.at[idx_v.at[pl.ds(j * _CH, _CH)]], bufs[b], gs[b])

        def write_out(j, b):
            return pltpu.make_async_copy(
                bufs[b], out_hbm.at[pl.ds(base + j * _CH, _CH)], os[b])

        gather_in(0, 0).start()
        if nch > 1:
            gather_in(1, 1).start()
        for j in range(nch):
            b = j % 2
            gather_in(j, b).wait()
            write_out(j, b).start()
            if j + 2 < nch:
                write_out(j, b).wait()
                gather_in(j + 2, b).start()
        for j in range(max(nch - 2, 0), nch):
            write_out(j, j % 2).wait()

    return gather_kernel(table, idx)


def _tc_project(x, wt, bias, tok_base, prev):
    bm = 512
    tpc = x.shape[0]
    base = tok_base // bm

    nk = 4
    kw = _D // nk

    def mm_kernel(*refs):
        x_ref, wt_ref, b_ref = refs[0], refs[1], refs[2]
        o_ref = refs[-1]
        acc = None
        for t in range(nk):
            xb = x_ref[:, t * kw:(t + 1) * kw].astype(jnp.bfloat16)
            p = jnp.dot(xb, wt_ref[t * kw:(t + 1) * kw, :],
                        preferred_element_type=jnp.float32)
            acc = p if acc is None else acc + p
        o_ref[...] = acc + b_ref[...]

    in_specs = [
        pl.BlockSpec((bm, _D), lambda i: (i, 0)),
        pl.BlockSpec((_D, _D), lambda i: (0, 0)),
        pl.BlockSpec((1, _D), lambda i: (0, 0)),
    ]
    args = [x, wt, bias]
    kwargs = {}
    if prev is not None:
        in_specs.append(pl.BlockSpec(memory_space=pltpu.MemorySpace.HBM))
        args.append(prev)
        kwargs["input_output_aliases"] = {3: 0}
    return pl.pallas_call(
        mm_kernel,
        grid=(tpc // bm,),
        in_specs=in_specs,
        out_specs=pl.BlockSpec((bm, _D), lambda i: (base + i, 0)),
        out_shape=jax.ShapeDtypeStruct((_NTOK, _D), jnp.float32),
        **kwargs,
    )(*args)


def kernel(input_tensor, embedding_table, proj_W, proj_b):
    b, s = input_tensor.shape
    idx = input_tensor.reshape(-1).astype(jnp.int32)
    wt = proj_W.T.astype(jnp.bfloat16)
    bias = proj_b.reshape(1, -1)
    out = None
    tok_base = 0
    for tpc in _CHUNKS:
        g = _sc_gather(embedding_table, idx[tok_base:tok_base + tpc], tpc)
        out = _tc_project(g, wt, bias, tok_base, out)
        tok_base += tpc
    return out.reshape(b, s, -1)


# speedup vs baseline: 1.0324x; 1.0324x over previous
import functools

import jax
import jax.numpy as jnp
from jax import lax
from jax.experimental import pallas as pl
from jax.experimental.pallas import tpu as pltpu
from jax.experimental.pallas import tpu_sc as plsc

_VOCAB = 32000
_D = 2048
_NTOK = 4 * 2048

_NC, _NS = 2, 16
_NW = _NC * _NS

_CHUNKS = (3584, 4608)
_CH = 16
_NBUF = 3


def _sc_gather(table, idx, tpc):
    mesh = plsc.VectorSubcoreMesh(core_axis_name="c", subcore_axis_name="s")
    b_per_w = tpc // _NW
    nch = b_per_w // _CH

    @functools.partial(
        pl.kernel,
        mesh=mesh,
        out_type=jax.ShapeDtypeStruct((tpc, _D), jnp.float32),
        scratch_types=(
            [pltpu.VMEM((b_per_w,), jnp.int32)]
            + [pltpu.VMEM((_CH, _D), jnp.float32) for _ in range(_NBUF)]
            + [pltpu.SemaphoreType.DMA for _ in range(2 * _NBUF)]
        ),
    )
    def gather_kernel(table_hbm, idx_hbm, out_hbm, idx_v, *scratch):
        bufs = scratch[:_NBUF]
        gs = scratch[_NBUF:2 * _NBUF]
        os = scratch[2 * _NBUF:]
        wid = lax.axis_index("s") * _NC + lax.axis_index("c")
        base = wid * b_per_w
        pltpu.sync_copy(idx_hbm.at[pl.ds(base, b_per_w)], idx_v)

        def gather_in(j, b):
            return pltpu.make_async_copy(
                table_hbm.at[idx_v.at[pl.ds(j * _CH, _CH)]], bufs[b], gs[b])

        def write_out(j, b):
            return pltpu.make_async_copy(
                bufs[b], out_hbm.at[pl.ds(base + j * _CH, _CH)], os[b])

        for b in range(min(_NBUF, nch)):
            gather_in(b, b).start()
        for j in range(nch):
            b = j % _NBUF
            gather_in(j, b).wait()
            write_out(j, b).start()
            if j + _NBUF < nch:
                write_out(j, b).wait()
                gather_in(j + _NBUF, b).start()
        for j in range(max(nch - _NBUF, 0), nch):
            write_out(j, j % _NBUF).wait()

    return gather_kernel(table, idx)


def _tc_project(x, wt, bias, tok_base, prev):
    bm = 512
    tpc = x.shape[0]
    base = tok_base // bm

    nk = 4
    kw = _D // nk

    def mm_kernel(*refs):
        x_ref, wt_ref, b_ref = refs[0], refs[1], refs[2]
        o_ref = refs[-1]
        acc = None
        for t in range(nk):
            xb = x_ref[:, t * kw:(t + 1) * kw].astype(jnp.bfloat16)
            p = jnp.dot(xb, wt_ref[t * kw:(t + 1) * kw, :],
                        preferred_element_type=jnp.float32)
            acc = p if acc is None else acc + p
        o_ref[...] = acc + b_ref[...]

    in_specs = [
        pl.BlockSpec((bm, _D), lambda i: (i, 0)),
        pl.BlockSpec((_D, _D), lambda i: (0, 0)),
        pl.BlockSpec((1, _D), lambda i: (0, 0)),
    ]
    args = [x, wt, bias]
    kwargs = {}
    if prev is not None:
        in_specs.append(pl.BlockSpec(memory_space=pltpu.MemorySpace.HBM))
        args.append(prev)
        kwargs["input_output_aliases"] = {3: 0}
    return pl.pallas_call(
        mm_kernel,
        grid=(tpc // bm,),
        in_specs=in_specs,
        out_specs=pl.BlockSpec((bm, _D), lambda i: (base + i, 0)),
        out_shape=jax.ShapeDtypeStruct((_NTOK, _D), jnp.float32),
        **kwargs,
    )(*args)


def kernel(input_tensor, embedding_table, proj_W, proj_b):
    b, s = input_tensor.shape
    idx = input_tensor.reshape(-1).astype(jnp.int32)
    wt = proj_W.T.astype(jnp.bfloat16)
    bias = proj_b.reshape(1, -1)
    out = None
    tok_base = 0
    for tpc in _CHUNKS:
        g = _sc_gather(embedding_table, idx[tok_base:tok_base + tpc], tpc)
        out = _tc_project(g, wt, bias, tok_base, out)
        tok_base += tpc
    return out.reshape(b, s, -1)
